# Initial kernel scaffold; baseline (speedup 1.0000x reference)
import functools

import jax
import jax.numpy as jnp
from jax import lax
from jax.experimental import pallas as pl
from jax.experimental.pallas import tpu as pltpu
from jax.experimental.pallas import tpu_sc as plsc

N = 50000
E = 800000
D = 64
G = 512
NUM_ATOMIC = 119
NUM_HYB = 8
NUM_AROM = 2
NUM_CHIR = 4
NUM_CHG = 9
NUM_BOND = 22

NS = 16
C = 1024
KROW = C // 128
NCH = 52
PER_TILE = NCH * C
EPP = NS * PER_TILE
PAD = EPP - (E + N)
NPT = N // NS

A_ROWS = NUM_BOND + NUM_ATOMIC + 1
B_ROWS = NUM_ATOMIC + 1 + 1

BN = 2000
BE = 2048


def _prep_nodes_body(x_ref, wn_ref, wh_ref, wa_ref, wc_ref, wg_ref, o_ref):
    xb = x_ref[...]
    def oh_dot(col, w_ref, rows):
        idx = xb[:, col].reshape(BN, 1)
        oh = (idx == lax.broadcasted_iota(jnp.int32, (1, rows), 1)).astype(jnp.float32)
        return jnp.dot(oh, w_ref[...], preferred_element_type=jnp.float32)
    emb = (oh_dot(0, wn_ref, NUM_ATOMIC) + oh_dot(1, wh_ref, NUM_HYB)
           + oh_dot(2, wa_ref, NUM_AROM) + oh_dot(3, wc_ref, NUM_CHIR)
           + oh_dot(4, wg_ref, NUM_CHG))
    o_ref[0] = emb[:, :32]
    o_ref[1] = emb[:, 32:]


def _prep_edges_body(ia_ref, ib_ref, a_ref, b_ref, o_ref):
    ia = ia_ref[0, 0, :].reshape(BE, 1)
    ib = ib_ref[0, 0, :].reshape(BE, 1)
    oha = (ia == lax.broadcasted_iota(jnp.int32, (1, A_ROWS), 1)).astype(jnp.float32)
    ohb = (ib == lax.broadcasted_iota(jnp.int32, (1, B_ROWS), 1)).astype(jnp.float32)
    emb = (jnp.dot(oha, a_ref[...], preferred_element_type=jnp.float32)
           + jnp.dot(ohb, b_ref[...], preferred_element_type=jnp.float32))
    o_ref[0] = emb[:, :32]
    o_ref[1] = emb[:, 32:]


def _mlp_pool_body(h_ref, batch_ref, w1_ref, b1_ref, w2_ref, b2_ref, w3_ref, o_ref):
    i = pl.program_id(0)
    h = jnp.concatenate([h_ref[0], h_ref[1]], axis=1)
    h = jnp.maximum(h, 0.0)
    h = lax.dot_general(h, w1_ref[...], (((1,), (1,)), ((), ())),
                        preferred_element_type=jnp.float32) + b1_ref[...]
    h = jnp.maximum(h, 0.0)
    h = lax.dot_general(h, w2_ref[...], (((1,), (1,)), ((), ())),
                        preferred_element_type=jnp.float32) + b2_ref[...]
    h = jnp.maximum(h, 0.0)
    energy = lax.dot_general(h, w3_ref[...], (((1,), (1,)), ((), ())),
                             preferred_element_type=jnp.float32)
    bidx = batch_ref[0, 0, :].reshape(BN, 1)
    oh = (bidx == lax.broadcasted_iota(jnp.int32, (1, G), 1)).astype(jnp.float32)
    part = lax.dot_general(oh, energy, (((0,), (0,)), ((), ())),
                           preferred_element_type=jnp.float32)
    @pl.when(i == 0)
    def _():
        o_ref[...] = jnp.zeros_like(o_ref)
    o_ref[...] += part


def _mp_layer_body(node_hbm, src_hbm, dst_hbm, edge_hbm, zrow_hbm, out_hbm,
                   srcv, dstv, nrows, erows, acc, sem):
    c = lax.axis_index("c")
    s = lax.axis_index("s")
    pltpu.sync_copy(zrow_hbm, acc.at[pl.ds(s * NPT, NPT)])
    plsc.subcore_barrier()

    def chunk(k, carry):
        e0 = s * PER_TILE + k * C
        r0 = s * (PER_TILE // 128) + k * KROW
        pltpu.sync_copy(src_hbm.at[c, pl.ds(r0, KROW)], srcv)
        pltpu.sync_copy(dst_hbm.at[pl.ds(r0, KROW)], dstv)
        cps = [pltpu.async_copy(node_hbm.at[srcv.at[j]],
                                nrows.at[pl.ds(j * 128, 128)], sem)
               for j in range(KROW)]
        pltpu.sync_copy(edge_hbm.at[c, pl.ds(e0, C)], erows)
        for cp in cps:
            cp.wait()

        def mul(i, carry2):
            nrows[i, pl.ds(0, 16)] = nrows[i, pl.ds(0, 16)] * erows[i, pl.ds(0, 16)]
            nrows[i, pl.ds(16, 16)] = nrows[i, pl.ds(16, 16)] * erows[i, pl.ds(16, 16)]
            return carry2
        lax.fori_loop(0, C, mul, 0, unroll=4)

        for j in range(KROW):
            pltpu.sync_copy(nrows.at[pl.ds(j * 128, 128)],
                            acc.at[dstv.at[j]], add=True)
        return carry
    lax.fori_loop(0, NCH, chunk, 0)
    plsc.subcore_barrier()
    pltpu.sync_copy(acc.at[pl.ds(s * NPT, NPT)],
                    out_hbm.at[c, pl.ds(s * NPT, NPT)])


_mp_layer = functools.partial(
    pl.kernel,
    out_type=jax.ShapeDtypeStruct((2, N, 32), jnp.float32),
    mesh=plsc.VectorSubcoreMesh(core_axis_name="c", subcore_axis_name="s"),
    scratch_types=[
        pltpu.VMEM((KROW, 128), jnp.int32),
        pltpu.VMEM((KROW, 128), jnp.int32),
        pltpu.VMEM((C, 32), jnp.float32),
        pltpu.VMEM((C, 32), jnp.float32),
        pltpu.VMEM_SHARED((N, 32), jnp.float32),
        pltpu.SemaphoreType.DMA,
    ],
)(_mp_layer_body)


def kernel(x, edge_index, edge_attr, batch, W_num, W_hyb, W_arom, W_chir, W_chg,
           W_bt, W_bi, W_slat, W_sl, W1, b1, W2, b2, W3):
    x = x.astype(jnp.int32)
    ei = edge_index.astype(jnp.int32)
    ea = edge_attr.astype(jnp.int32)
    batch = batch.astype(jnp.int32)

    loop = jnp.arange(N, dtype=jnp.int32)
    zpad = jnp.zeros((PAD,), jnp.int32)
    src = jnp.concatenate([ei[0], loop, zpad])
    dst = jnp.concatenate([ei[1], loop, zpad])
    src01 = jnp.stack([src, src + N]).reshape(2, EPP // 128, 128)
    dst2d = dst.reshape(EPP // 128, 128)

    ia = jnp.concatenate([ea[:, 0], NUM_BOND + x[:, 0],
                          jnp.full((PAD,), A_ROWS - 1, jnp.int32)]).reshape(EPP // BE, 1, BE)
    ib = jnp.concatenate([ea[:, 1], jnp.full((N,), NUM_ATOMIC, jnp.int32),
                          jnp.full((PAD,), B_ROWS - 1, jnp.int32)]).reshape(EPP // BE, 1, BE)
    A = jnp.concatenate([W_bt, W_slat, jnp.zeros((1, D), jnp.float32)], axis=0)
    B = jnp.concatenate([W_bi, W_sl, jnp.zeros((1, D), jnp.float32)], axis=0)
    zrows = jnp.zeros((NPT, 32), jnp.float32)

    node0 = pl.pallas_call(
        _prep_nodes_body,
        grid=(N // BN,),
        in_specs=[
            pl.BlockSpec((BN, 5), lambda i: (i, 0)),
            pl.BlockSpec((NUM_ATOMIC, D), lambda i: (0, 0)),
            pl.BlockSpec((NUM_HYB, D), lambda i: (0, 0)),
            pl.BlockSpec((NUM_AROM, D), lambda i: (0, 0)),
            pl.BlockSpec((NUM_CHIR, D), lambda i: (0, 0)),
            pl.BlockSpec((NUM_CHG, D), lambda i: (0, 0)),
        ],
        out_specs=pl.BlockSpec((2, BN, 32), lambda i: (0, i, 0)),
        out_shape=jax.ShapeDtypeStruct((2, N, 32), jnp.float32),
    )(x, W_num, W_hyb, W_arom, W_chir, W_chg)

    edge_tab = pl.pallas_call(
        _prep_edges_body,
        grid=(EPP // BE,),
        in_specs=[
            pl.BlockSpec((1, 1, BE), lambda i: (i, 0, 0)),
            pl.BlockSpec((1, 1, BE), lambda i: (i, 0, 0)),
            pl.BlockSpec((A_ROWS, D), lambda i: (0, 0)),
            pl.BlockSpec((B_ROWS, D), lambda i: (0, 0)),
        ],
        out_specs=pl.BlockSpec((2, BE, 32), lambda i: (0, i, 0)),
        out_shape=jax.ShapeDtypeStruct((2, EPP, 32), jnp.float32),
    )(ia, ib, A, B)

    node1 = _mp_layer(node0.reshape(2 * N, 32), src01, dst2d, edge_tab, zrows)
    node2 = _mp_layer(node1.reshape(2 * N, 32), src01, dst2d, edge_tab, zrows)

    dg = pl.pallas_call(
        _mlp_pool_body,
        grid=(N // BN,),
        in_specs=[
            pl.BlockSpec((2, BN, 32), lambda i: (0, i, 0)),
            pl.BlockSpec((1, 1, BN), lambda i: (i, 0, 0)),
            pl.BlockSpec((D, D), lambda i: (0, 0)),
            pl.BlockSpec((1, D), lambda i: (0, 0)),
            pl.BlockSpec((D // 2, D), lambda i: (0, 0)),
            pl.BlockSpec((1, D // 2), lambda i: (0, 0)),
            pl.BlockSpec((1, D // 2), lambda i: (0, 0)),
        ],
        out_specs=pl.BlockSpec((G, 1), lambda i: (0, 0)),
        out_shape=jax.ShapeDtypeStruct((G, 1), jnp.float32),
    )(node2, batch.reshape(N // BN, 1, BN), W1, b1.reshape(1, D),
      W2, b2.reshape(1, D // 2), W3)
    return dg


# speedup vs baseline: 3.8860x; 3.8860x over previous
import functools

import jax
import jax.numpy as jnp
from jax import lax
from jax.experimental import pallas as pl
from jax.experimental.pallas import tpu as pltpu
from jax.experimental.pallas import tpu_sc as plsc

N = 50000
E = 800000
D = 64
G = 512
NUM_ATOMIC = 119
NUM_HYB = 8
NUM_AROM = 2
NUM_CHIR = 4
NUM_CHG = 9
NUM_BOND = 22

NS = 16
C = 1024
KROW = C // 128
NCH = 52
PER_TILE = NCH * C
EPP = NS * PER_TILE
PAD = EPP - (E + N)
NROW = 50048
NPT = NROW // NS
QW = 16

A_ROWS = NUM_BOND + NUM_ATOMIC + 1
B_ROWS = NUM_ATOMIC + 1 + 1

BN = 2000
BE = 2048


def _prep_nodes_body(x_ref, wn_ref, wh_ref, wa_ref, wc_ref, wg_ref, o_ref):
    xb = x_ref[...]
    def oh_dot(col, w_ref, rows):
        idx = xb[:, col].reshape(BN, 1)
        oh = (idx == lax.broadcasted_iota(jnp.int32, (1, rows), 1)).astype(jnp.float32)
        return jnp.dot(oh, w_ref[...], preferred_element_type=jnp.float32)
    emb = (oh_dot(0, wn_ref, NUM_ATOMIC) + oh_dot(1, wh_ref, NUM_HYB)
           + oh_dot(2, wa_ref, NUM_AROM) + oh_dot(3, wc_ref, NUM_CHIR)
           + oh_dot(4, wg_ref, NUM_CHG))
    for q in range(4):
        o_ref[q] = emb[:, q * QW:(q + 1) * QW]


def _prep_edges_body(ia_ref, ib_ref, a_ref, b_ref, o_ref):
    ia = ia_ref[0, 0, :].reshape(BE, 1)
    ib = ib_ref[0, 0, :].reshape(BE, 1)
    oha = (ia == lax.broadcasted_iota(jnp.int32, (1, A_ROWS), 1)).astype(jnp.float32)
    ohb = (ib == lax.broadcasted_iota(jnp.int32, (1, B_ROWS), 1)).astype(jnp.float32)
    emb = (jnp.dot(oha, a_ref[...], preferred_element_type=jnp.float32)
           + jnp.dot(ohb, b_ref[...], preferred_element_type=jnp.float32))
    for q in range(4):
        o_ref[q] = emb[:, q * QW:(q + 1) * QW]


def _mlp_pool_body(h_ref, batch_ref, w1_ref, b1_ref, w2_ref, b2_ref, w3_ref, o_ref):
    i = pl.program_id(0)
    h = jnp.concatenate([h_ref[0], h_ref[1], h_ref[2], h_ref[3]], axis=1)
    h = jnp.maximum(h, 0.0)
    h = lax.dot_general(h, w1_ref[...], (((1,), (1,)), ((), ())),
                        preferred_element_type=jnp.float32) + b1_ref[...]
    h = jnp.maximum(h, 0.0)
    h = lax.dot_general(h, w2_ref[...], (((1,), (1,)), ((), ())),
                        preferred_element_type=jnp.float32) + b2_ref[...]
    h = jnp.maximum(h, 0.0)
    energy = lax.dot_general(h, w3_ref[...], (((1,), (1,)), ((), ())),
                             preferred_element_type=jnp.float32)
    bidx = batch_ref[0, 0, :].reshape(BN, 1)
    oh = (bidx == lax.broadcasted_iota(jnp.int32, (1, G), 1)).astype(jnp.float32)
    part = lax.dot_general(oh, energy, (((0,), (0,)), ((), ())),
                           preferred_element_type=jnp.float32)
    @pl.when(i == 0)
    def _():
        o_ref[...] = jnp.zeros_like(o_ref)
    o_ref[...] += part


def _mp_layer_body(node_hbm, src_hbm, dst_hbm, edge_hbm, zrow_hbm, out_hbm,
                   srcv, dstv, nrows, erows, acc, sem):
    c = lax.axis_index("c")
    s = lax.axis_index("s")
    for p in range(2):
        q = 2 * c + p
        pltpu.sync_copy(zrow_hbm, acc.at[pl.ds(s * NPT, NPT)])
        plsc.subcore_barrier()

        def chunk(k, carry):
            e0 = s * PER_TILE + k * C
            r0 = s * (PER_TILE // 128) + k * KROW
            pltpu.sync_copy(src_hbm.at[q, pl.ds(r0, KROW)], srcv)
            pltpu.sync_copy(dst_hbm.at[pl.ds(r0, KROW)], dstv)
            cps = [pltpu.async_copy(node_hbm.at[srcv.at[j]],
                                    nrows.at[pl.ds(j * 128, 128)], sem)
                   for j in range(KROW)]
            pltpu.sync_copy(edge_hbm.at[q, pl.ds(e0, C)], erows)
            for cp in cps:
                cp.wait()

            def mul(i, carry2):
                nrows[i, pl.ds(0, QW)] = (nrows[i, pl.ds(0, QW)]
                                          * erows[i, pl.ds(0, QW)])
                return carry2
            lax.fori_loop(0, C, mul, 0, unroll=8)

            for j in range(KROW):
                pltpu.sync_copy(nrows.at[pl.ds(j * 128, 128)],
                                acc.at[dstv.at[j]], add=True)
            return carry
        lax.fori_loop(0, NCH, chunk, 0)
        plsc.subcore_barrier()
        pltpu.sync_copy(acc.at[pl.ds(s * NPT, NPT)],
                        out_hbm.at[q, pl.ds(s * NPT, NPT)])


_mp_layer = functools.partial(
    pl.kernel,
    out_type=jax.ShapeDtypeStruct((4, NROW, QW), jnp.float32),
    mesh=plsc.VectorSubcoreMesh(core_axis_name="c", subcore_axis_name="s"),
    scratch_types=[
        pltpu.VMEM((KROW, 128), jnp.int32),
        pltpu.VMEM((KROW, 128), jnp.int32),
        pltpu.VMEM((C, QW), jnp.float32),
        pltpu.VMEM((C, QW), jnp.float32),
        pltpu.VMEM_SHARED((NROW, QW), jnp.float32),
        pltpu.SemaphoreType.DMA,
    ],
    compiler_params=pltpu.CompilerParams(use_tc_tiling_on_sc=False),
)(_mp_layer_body)


def kernel(x, edge_index, edge_attr, batch, W_num, W_hyb, W_arom, W_chir, W_chg,
           W_bt, W_bi, W_slat, W_sl, W1, b1, W2, b2, W3):
    x = x.astype(jnp.int32)
    ei = edge_index.astype(jnp.int32)
    ea = edge_attr.astype(jnp.int32)
    batch = batch.astype(jnp.int32)

    loop = jnp.arange(N, dtype=jnp.int32)
    zpad = jnp.zeros((PAD,), jnp.int32)
    src = jnp.concatenate([ei[0], loop, zpad])
    dst = jnp.concatenate([ei[1], loop, zpad])
    src4 = jnp.stack([src + qq * NROW for qq in range(4)]).reshape(4, EPP // 128, 128)
    dst2d = dst.reshape(EPP // 128, 128)

    ia = jnp.concatenate([ea[:, 0], NUM_BOND + x[:, 0],
                          jnp.full((PAD,), A_ROWS - 1, jnp.int32)]).reshape(EPP // BE, 1, BE)
    ib = jnp.concatenate([ea[:, 1], jnp.full((N,), NUM_ATOMIC, jnp.int32),
                          jnp.full((PAD,), B_ROWS - 1, jnp.int32)]).reshape(EPP // BE, 1, BE)
    A = jnp.concatenate([W_bt, W_slat, jnp.zeros((1, D), jnp.float32)], axis=0)
    B = jnp.concatenate([W_bi, W_sl, jnp.zeros((1, D), jnp.float32)], axis=0)
    zrows = jnp.zeros((NPT, QW), jnp.float32)

    node0 = pl.pallas_call(
        _prep_nodes_body,
        grid=(N // BN,),
        in_specs=[
            pl.BlockSpec((BN, 5), lambda i: (i, 0)),
            pl.BlockSpec((NUM_ATOMIC, D), lambda i: (0, 0)),
            pl.BlockSpec((NUM_HYB, D), lambda i: (0, 0)),
            pl.BlockSpec((NUM_AROM, D), lambda i: (0, 0)),
            pl.BlockSpec((NUM_CHIR, D), lambda i: (0, 0)),
            pl.BlockSpec((NUM_CHG, D), lambda i: (0, 0)),
        ],
        out_specs=pl.BlockSpec((4, BN, QW), lambda i: (0, i, 0)),
        out_shape=jax.ShapeDtypeStruct((4, NROW, QW), jnp.float32),
    )(x, W_num, W_hyb, W_arom, W_chir, W_chg)

    edge_tab = pl.pallas_call(
        _prep_edges_body,
        grid=(EPP // BE,),
        in_specs=[
            pl.BlockSpec((1, 1, BE), lambda i: (i, 0, 0)),
            pl.BlockSpec((1, 1, BE), lambda i: (i, 0, 0)),
            pl.BlockSpec((A_ROWS, D), lambda i: (0, 0)),
            pl.BlockSpec((B_ROWS, D), lambda i: (0, 0)),
        ],
        out_specs=pl.BlockSpec((4, BE, QW), lambda i: (0, i, 0)),
        out_shape=jax.ShapeDtypeStruct((4, EPP, QW), jnp.float32),
    )(ia, ib, A, B)

    node1 = _mp_layer(node0.reshape(4 * NROW, QW), src4, dst2d, edge_tab, zrows)
    node2 = _mp_layer(node1.reshape(4 * NROW, QW), src4, dst2d, edge_tab, zrows)

    dg = pl.pallas_call(
        _mlp_pool_body,
        grid=(N // BN,),
        in_specs=[
            pl.BlockSpec((4, BN, QW), lambda i: (0, i, 0)),
            pl.BlockSpec((1, 1, BN), lambda i: (i, 0, 0)),
            pl.BlockSpec((D, D), lambda i: (0, 0)),
            pl.BlockSpec((1, D), lambda i: (0, 0)),
            pl.BlockSpec((D // 2, D), lambda i: (0, 0)),
            pl.BlockSpec((1, D // 2), lambda i: (0, 0)),
            pl.BlockSpec((1, D // 2), lambda i: (0, 0)),
        ],
        out_specs=pl.BlockSpec((G, 1), lambda i: (0, 0)),
        out_shape=jax.ShapeDtypeStruct((G, 1), jnp.float32),
    )(node2, batch.reshape(N // BN, 1, BN), W1, b1.reshape(1, D),
      W2, b2.reshape(1, D // 2), W3)
    return dg
